# baseline (speedup 1.0000x reference)
import functools

import jax
import jax.numpy as jnp
from jax import lax
from jax.experimental import pallas as pl
from jax.experimental.pallas import tpu as pltpu
from jax.experimental.pallas import tpu_sc as plsc

NC = 2
NS = 16
NW = NC * NS
KSUB = 80
KB = 800
NSUB = KB // KSUB


def _node_pre(h, w_src, a_mat, att2):
    n = h.shape[0]
    c = w_src.shape[1]

    def body(h_ref, w_ref, a_ref, att_ref, p_ref, g_ref, ab_ref, abs_ref, mab_ref):
        hh = h_ref[...]
        p = jnp.dot(hh, w_ref[...], preferred_element_type=jnp.float32,
                    precision=lax.Precision.HIGHEST)
        p_ref[...] = p
        g_ref[...] = jnp.dot(p, a_ref[...], preferred_element_type=jnp.float32,
                             precision=lax.Precision.HIGHEST)
        att = att_ref[...]
        asrc = jnp.sum(p * att[0:1, :], axis=1)
        adst = jnp.sum(p * att[1:2, :], axis=1)
        ab_ref[...] = jnp.stack([asrc, adst])
        abs_ref[...] = (asrc + adst)[:, None]
        mab_ref[...] = (jnp.max(asrc) + jnp.max(adst)).reshape(1, 1)

    return pl.pallas_call(
        body,
        out_shape=[
            jax.ShapeDtypeStruct((n, c), jnp.float32),
            jax.ShapeDtypeStruct((n, c), jnp.float32),
            jax.ShapeDtypeStruct((2, n), jnp.float32),
            jax.ShapeDtypeStruct((n, 1), jnp.float32),
            jax.ShapeDtypeStruct((1, 1), jnp.float32),
        ],
    )(h, w_src, a_mat, att2)


def _edge_absmax(ea_rows):
    def body(ea_ref, out_ref):
        out_ref[...] = jnp.max(jnp.abs(ea_ref[...])).reshape(1, 1)

    return pl.pallas_call(
        body, out_shape=jax.ShapeDtypeStruct((1, 1), jnp.float32),
    )(ea_rows)


def _node_post(acc2, p, absum, la, wrow, bias, c, cw, first, relu):
    n = p.shape[0]

    def body(acc_ref, p_ref, abs_ref, la_ref, w_ref, b_ref, out_ref, la_out_ref=None):
        a = acc_ref[...]
        a = a[0, :n] + a[1, :n]
        msg = a[:, 0:c]
        s = a[:, c:c + 1]
        if first:
            cnt = a[:, c + 3:c + 4]
            ls = a[:, c + 1:c + 3]
            lav = jnp.where(cnt > 0, ls / jnp.maximum(cnt, 1.0), 0.0)
            la_out_ref[...] = lav
        else:
            lav = la_ref[...]
        w = w_ref[...]
        av = lav[:, 0:1] * w[0, 0] + lav[:, 1:2] * w[0, 1]
        al = abs_ref[...] + av
        al = jnp.maximum(al, 0.2 * al)
        exl = jnp.exp(al - w[0, 2])
        o = (msg + exl * p_ref[...]) / (s + exl + 1e-16) + b_ref[...]
        out_ref[...] = jnp.maximum(o, 0.0) if relu else o

    outs = [jax.ShapeDtypeStruct((n, c), jnp.float32)]
    if first:
        outs.append(jax.ShapeDtypeStruct((n, 2), jnp.float32))
    return pl.pallas_call(body, out_shape=outs)(acc2, p, absum, la, wrow, bias)


NPAD = 10240


@functools.lru_cache(maxsize=None)
def _make_sc(n, e, c, cw, first):
    epw = e // NW
    nout = epw // KB
    npt = NPAD // NS
    nch = c // 16
    mesh = plsc.VectorSubcoreMesh(core_axis_name="c", subcore_axis_name="s")

    nst = n // NS

    def body(ints_h, flts_h, g_h, asrc_h, adst_h, wv_h, z_h, out_h,
             acc_sp, g_sp, asrc_t, adst_t, ib, fb,
             grows0, grows1, grows2, msgs0, msgs1, msgs2,
             didx0, didx1, didx2, exb, wvb,
             gsem0, gsem1, gsem2, ssem0, ssem1, ssem2):
        grows_l = (grows0, grows1, grows2)
        msgs_l = (msgs0, msgs1, msgs2)
        didx_l = (didx0, didx1, didx2)
        gsem_l = (gsem0, gsem1, gsem2)
        ssem_l = (ssem0, ssem1, ssem2)
        cid = lax.axis_index("c")
        sid = lax.axis_index("s")
        wid = sid * NC + cid
        pltpu.sync_copy(asrc_h, asrc_t)
        pltpu.sync_copy(adst_h, adst_t)
        pltpu.sync_copy(wv_h, wvb)
        pltpu.sync_copy(z_h, acc_sp.at[pl.ds(sid * npt, npt)])
        pltpu.sync_copy(g_h.at[pl.ds(sid * nst, nst)],
                        g_sp.at[pl.ds(sid * nst, nst)])

        w3 = wvb[3, pl.ds(0, 16)]
        mv = w3[0]
        v0 = w3[1]
        v1 = w3[2]
        b0v = [wvb[0, pl.ds(16 * t, 16)] for t in range(nch)]
        b1v = [wvb[1, pl.ds(16 * t, 16)] for t in range(nch)]
        cvv = [wvb[2, pl.ds(16 * t, 16)] for t in range(nch)]
        b0s = [b0v[t // 16][t % 16] for t in range(c)]
        b1s = [b1v[t // 16][t % 16] for t in range(c)]
        cvs = [cvv[t // 16][t % 16] for t in range(c)]
        z16 = jnp.zeros((16,), jnp.float32)
        colv = [jnp.full((16,), t, jnp.int32) for t in range(c)]

        def zrow(r, _):
            for mref in msgs_l:
                for t in range(cw // 16):
                    mref[r, pl.ds(16 * t, 16)] = z16
            return 0
        lax.fori_loop(0, KSUB, zrow, 0)

        plsc.subcore_barrier()

        iot = lax.iota(jnp.int32, 16)
        col_c = jnp.full((16,), c, jnp.int32)
        one16 = jnp.full((16,), 1.0, jnp.float32)

        def fill_didx(q, dref):
            for r in range(KSUB // 16):
                dref[pl.ds(r * 16, 16)] = ib[1, pl.ds(q * KSUB + r * 16, 16)]

        def start_gather(q, bi):
            return pltpu.async_copy(g_sp.at[ib.at[0, pl.ds(q * KSUB, KSUB)]],
                                    grows_l[bi], gsem_l[bi])

        def start_scatter(bi):
            return pltpu.async_copy(msgs_l[bi], acc_sp.at[didx_l[bi]],
                                    ssem_l[bi], add=True)

        def compute(q, bi):
            grows = grows_l[bi]
            msgs = msgs_l[bi]
            didx = didx_l[bi]

            def a16(i, _):
                k0g = q * KSUB + i * 16
                k0 = i * 16
                s16 = ib[0, pl.ds(k0g, 16)]
                d16 = didx[pl.ds(k0, 16)]
                a1 = plsc.load_gather(asrc_t, [s16])
                a2 = plsc.load_gather(adst_t, [d16])
                e0 = fb[2, pl.ds(k0g, 16)]
                e1 = fb[3, pl.ds(k0g, 16)]
                al = a1 + a2 + e0 * v0 + e1 * v1
                al = jnp.maximum(al, 0.2 * al)
                ex = jnp.exp(al - mv)
                exb[pl.ds(k0, 16)] = ex
                rows = iot + k0
                plsc.store_scatter(msgs, [rows, col_c], ex)
                if first:
                    plsc.store_scatter(msgs, [rows, col_c + 1], e0)
                    plsc.store_scatter(msgs, [rows, col_c + 2], e1)
                    plsc.store_scatter(msgs, [rows, col_c + 3], one16)
                return 0
            lax.fori_loop(0, KSUB // 16, a16, 0)

            def pe(i, _):
                k0g = q * KSUB + i * 16
                k0 = i * 16
                rows = iot + k0
                f0v = fb[0, pl.ds(k0g, 16)]
                f1v = fb[1, pl.ds(k0g, 16)]
                exv = exb[pl.ds(k0, 16)]
                for t in range(c):
                    gcol = plsc.load_gather(grows, [rows, colv[t]])
                    m = jnp.maximum(gcol + f0v * b0s[t] + f1v * b1s[t] + cvs[t],
                                    0.0)
                    plsc.store_scatter(msgs, [rows, colv[t]], m * exv)
                return 0
            lax.fori_loop(0, KSUB // 16, pe, 0)

        def outer(j, _):
            brow = wid * nout + j
            pltpu.sync_copy(ints_h.at[brow], ib)
            pltpu.sync_copy(flts_h.at[brow], fb)

            gd = {}
            sd = {}
            fill_didx(0, didx_l[0])
            gd[0] = start_gather(0, 0)
            for q in range(NSUB):
                cur = q % 3
                if q >= 2:
                    sd[q - 2].wait()
                if q + 1 < NSUB:
                    nxt = (q + 1) % 3
                    fill_didx(q + 1, didx_l[nxt])
                    gd[q + 1] = start_gather(q + 1, nxt)
                gd[q].wait()
                compute(q, cur)
                sd[q] = start_scatter(cur)
            sd[NSUB - 2].wait()
            sd[NSUB - 1].wait()
            return 0
        lax.fori_loop(0, nout, outer, 0)

        plsc.subcore_barrier()
        pltpu.sync_copy(acc_sp.at[pl.ds(sid * npt, npt)],
                        out_h.at[cid, pl.ds(sid * npt, npt)])

    return pl.kernel(
        body,
        out_type=jax.ShapeDtypeStruct((NC, NPAD, cw), jnp.float32),
        mesh=mesh,
        compiler_params=pltpu.CompilerParams(needs_layout_passes=False,
                                             use_tc_tiling_on_sc=False),
        scratch_types=[
            pltpu.VMEM_SHARED((NPAD, cw), jnp.float32),
            pltpu.VMEM_SHARED((n, c), jnp.float32),
            pltpu.VMEM((n,), jnp.float32),
            pltpu.VMEM((n,), jnp.float32),
            pltpu.VMEM((2, KB), jnp.int32),
            pltpu.VMEM((4, KB), jnp.float32),
            pltpu.VMEM((KSUB, c), jnp.float32),
            pltpu.VMEM((KSUB, c), jnp.float32),
            pltpu.VMEM((KSUB, c), jnp.float32),
            pltpu.VMEM((KSUB, cw), jnp.float32),
            pltpu.VMEM((KSUB, cw), jnp.float32),
            pltpu.VMEM((KSUB, cw), jnp.float32),
            pltpu.VMEM((KSUB,), jnp.int32),
            pltpu.VMEM((KSUB,), jnp.int32),
            pltpu.VMEM((KSUB,), jnp.int32),
            pltpu.VMEM((KSUB,), jnp.float32),
            pltpu.VMEM((8, 48), jnp.float32),
            pltpu.SemaphoreType.DMA,
            pltpu.SemaphoreType.DMA,
            pltpu.SemaphoreType.DMA,
            pltpu.SemaphoreType.DMA,
            pltpu.SemaphoreType.DMA,
            pltpu.SemaphoreType.DMA,
        ],
    )


def kernel(x, edge_index, edge_attr, edge_feature, params):
    n = x.shape[0]
    e = edge_attr.shape[0]
    nblk = e // KB
    ints = jnp.stack([edge_index[0].reshape(nblk, KB),
                      edge_index[1].reshape(nblk, KB)], axis=1)
    flts = jnp.stack([edge_feature[:, 0].reshape(nblk, KB),
                      edge_feature[:, 1].reshape(nblk, KB),
                      edge_attr[:, 0].reshape(nblk, KB),
                      edge_attr[:, 1].reshape(nblk, KB)], axis=1)
    mea = _edge_absmax(edge_attr.reshape(-1, 512))[0, 0]

    h = x
    la = jnp.zeros((n, 2), jnp.float32)
    for li, name in enumerate(('mp1', 'mp2', 'mp3')):
        p = params[name]
        first = li == 0
        relu = li < 2
        c = p['att_src'].shape[-1]
        cw = 48 if c == 32 else 32
        a_mat = p['W_en'][:c, :]
        b8 = p['W_en'][c:, :]
        bf = p['W_ef'] @ b8
        cvec = p['b_ef'] @ b8 + p['b_en']
        v = p['W_edge'] @ p['att_edge'][0, 0, :]
        att2 = jnp.stack([p['att_src'][0, 0, :], p['att_dst'][0, 0, :]])

        pmat, gmat, ab, absum, mab = _node_pre(h, p['W_src'], a_mat, att2)
        m = jnp.maximum(mab[0, 0] + mea * (jnp.abs(v[0]) + jnp.abs(v[1])), 0.0)
        wv = (jnp.zeros((8, 48), jnp.float32)
              .at[0, :c].set(bf[0])
              .at[1, :c].set(bf[1])
              .at[2, :c].set(cvec)
              .at[3, 0].set(m)
              .at[3, 1].set(v[0])
              .at[3, 2].set(v[1]))
        zblk = jnp.zeros((NPAD // NS, cw), jnp.float32)

        acc2 = _make_sc(n, e, c, cw, first)(
            ints, flts, gmat, ab[0], ab[1], wv, zblk)

        wrow = jnp.zeros((1, 8), jnp.float32).at[0, 0].set(v[0]).at[0, 1].set(v[1]).at[0, 2].set(m)
        bias = p['bias'].reshape(1, c)
        res = _node_post(acc2, pmat, absum, la, wrow, bias, c, cw, first, relu)
        if first:
            h, la = res
        else:
            h = res[0]
    return h


# speedup vs baseline: 1.2675x; 1.2675x over previous
import functools

import jax
import jax.numpy as jnp
from jax import lax
from jax.experimental import pallas as pl
from jax.experimental.pallas import tpu as pltpu
from jax.experimental.pallas import tpu_sc as plsc

NC = 2
NS = 16
NW = NC * NS
KSUB = 80
KB = 800
NSUB = KB // KSUB


def _node_pre(h, w_src, a_mat, att2):
    n = h.shape[0]
    c = w_src.shape[1]

    def body(h_ref, w_ref, a_ref, att_ref, p_ref, g_ref, ab_ref, abs_ref, mab_ref):
        hh = h_ref[...]
        p = jnp.dot(hh, w_ref[...], preferred_element_type=jnp.float32,
                    precision=lax.Precision.HIGHEST)
        p_ref[...] = p
        g_ref[:n, :c] = jnp.dot(p, a_ref[...], preferred_element_type=jnp.float32,
                                precision=lax.Precision.HIGHEST)
        att = att_ref[...]
        asrc = jnp.sum(p * att[0:1, :], axis=1)
        adst = jnp.sum(p * att[1:2, :], axis=1)
        ab_ref[...] = jnp.stack([asrc, adst])
        abs_ref[...] = (asrc + adst)[:, None]
        mab_ref[...] = (jnp.max(asrc) + jnp.max(adst)).reshape(1, 1)

    return pl.pallas_call(
        body,
        out_shape=[
            jax.ShapeDtypeStruct((n, c), jnp.float32),
            jax.ShapeDtypeStruct((NPAD, c + 1), jnp.float32),
            jax.ShapeDtypeStruct((2, n), jnp.float32),
            jax.ShapeDtypeStruct((n, 1), jnp.float32),
            jax.ShapeDtypeStruct((1, 1), jnp.float32),
        ],
    )(h, w_src, a_mat, att2)


def _edge_absmax(ea_rows):
    def body(ea_ref, out_ref):
        out_ref[...] = jnp.max(jnp.abs(ea_ref[...])).reshape(1, 1)

    return pl.pallas_call(
        body, out_shape=jax.ShapeDtypeStruct((1, 1), jnp.float32),
    )(ea_rows)


def _node_post(acc2, p, absum, la, wrow, bias, c, cw, first, relu):
    n = p.shape[0]

    def body(acc_ref, p_ref, abs_ref, la_ref, w_ref, b_ref, out_ref, la_out_ref=None):
        a = acc_ref[...]
        a = a[0, :n] + a[1, :n]
        msg = a[:, 0:c]
        s = a[:, c:c + 1]
        if first:
            cnt = a[:, c + 3:c + 4]
            ls = a[:, c + 1:c + 3]
            lav = jnp.where(cnt > 0, ls / jnp.maximum(cnt, 1.0), 0.0)
            la_out_ref[...] = lav
        else:
            lav = la_ref[...]
        w = w_ref[...]
        av = lav[:, 0:1] * w[0, 0] + lav[:, 1:2] * w[0, 1]
        al = abs_ref[...] + av
        al = jnp.maximum(al, 0.2 * al)
        exl = jnp.exp(al - w[0, 2])
        o = (msg + exl * p_ref[...]) / (s + exl + 1e-16) + b_ref[...]
        out_ref[...] = jnp.maximum(o, 0.0) if relu else o

    outs = [jax.ShapeDtypeStruct((n, c), jnp.float32)]
    if first:
        outs.append(jax.ShapeDtypeStruct((n, 2), jnp.float32))
    return pl.pallas_call(body, out_shape=outs)(acc2, p, absum, la, wrow, bias)


NPAD = 10240


@functools.lru_cache(maxsize=None)
def _make_sc(n, e, c, first):
    cw = c + 17
    gp = c + 1
    epw = e // NW
    nout = epw // KB
    npt = NPAD // NS
    nch = c // 16
    mesh = plsc.VectorSubcoreMesh(core_axis_name="c", subcore_axis_name="s")

    nst = NPAD // NS

    def body(ints_h, flts_h, g_h, asrc_h, adst_h, wv_h, z_h, out_h,
             acc_sp, g_sp, asrc_t, adst_t, ib, fb,
             grows0, grows1, grows2, msgs0, msgs1, msgs2,
             didx0, didx1, didx2, exb, wvb,
             gsem0, gsem1, gsem2, ssem0, ssem1, ssem2):
        grows_l = (grows0, grows1, grows2)
        msgs_l = (msgs0, msgs1, msgs2)
        didx_l = (didx0, didx1, didx2)
        gsem_l = (gsem0, gsem1, gsem2)
        ssem_l = (ssem0, ssem1, ssem2)
        cid = lax.axis_index("c")
        sid = lax.axis_index("s")
        wid = sid * NC + cid
        pltpu.sync_copy(asrc_h, asrc_t)
        pltpu.sync_copy(adst_h, adst_t)
        pltpu.sync_copy(wv_h, wvb)
        pltpu.sync_copy(z_h, acc_sp.at[pl.ds(sid * npt, npt)])
        pltpu.sync_copy(g_h.at[pl.ds(sid * nst, nst)],
                        g_sp.at[pl.ds(sid * nst, nst)])

        w3 = wvb[3, pl.ds(0, 16)]
        mv = w3[0]
        v0 = w3[1]
        v1 = w3[2]
        b0v = [wvb[0, pl.ds(16 * t, 16)] for t in range(nch)]
        b1v = [wvb[1, pl.ds(16 * t, 16)] for t in range(nch)]
        cvv = [wvb[2, pl.ds(16 * t, 16)] for t in range(nch)]
        b0s = [b0v[t // 16][t % 16] for t in range(c)]
        b1s = [b1v[t // 16][t % 16] for t in range(c)]
        cvs = [cvv[t // 16][t % 16] for t in range(c)]
        z16 = jnp.zeros((16,), jnp.float32)
        colv = [jnp.full((16,), t, jnp.int32) for t in range(c)]

        def zrow(r, _):
            for mref in msgs_l:
                for t in range(c // 16):
                    mref[r, pl.ds(16 * t, 16)] = z16
                mref[r, pl.ds(c + 1, 16)] = z16
            return 0
        lax.fori_loop(0, KSUB, zrow, 0)

        plsc.subcore_barrier()

        iot = lax.iota(jnp.int32, 16)
        col_c = jnp.full((16,), c, jnp.int32)
        one16 = jnp.full((16,), 1.0, jnp.float32)

        def fill_didx(q, dref):
            for r in range(KSUB // 16):
                dref[pl.ds(r * 16, 16)] = ib[1, pl.ds(q * KSUB + r * 16, 16)]

        def start_gather(q, bi):
            return pltpu.async_copy(g_sp.at[ib.at[0, pl.ds(q * KSUB, KSUB)]],
                                    grows_l[bi], gsem_l[bi])

        def start_scatter(bi):
            return pltpu.async_copy(msgs_l[bi], acc_sp.at[didx_l[bi]],
                                    ssem_l[bi], add=True)

        def compute(q, bi):
            grows = grows_l[bi]
            msgs = msgs_l[bi]
            didx = didx_l[bi]

            def a16(i, _):
                k0g = q * KSUB + i * 16
                k0 = i * 16
                s16 = ib[0, pl.ds(k0g, 16)]
                d16 = didx[pl.ds(k0, 16)]
                a1 = plsc.load_gather(asrc_t, [s16])
                a2 = plsc.load_gather(adst_t, [d16])
                e0 = fb[2, pl.ds(k0g, 16)]
                e1 = fb[3, pl.ds(k0g, 16)]
                al = a1 + a2 + e0 * v0 + e1 * v1
                al = jnp.maximum(al, 0.2 * al)
                ex = jnp.exp(al - mv)
                exb[pl.ds(k0, 16)] = ex
                rows = iot + k0
                plsc.store_scatter(msgs, [rows, col_c], ex)
                if first:
                    plsc.store_scatter(msgs, [rows, col_c + 1], e0)
                    plsc.store_scatter(msgs, [rows, col_c + 2], e1)
                    plsc.store_scatter(msgs, [rows, col_c + 3], one16)
                return 0
            lax.fori_loop(0, KSUB // 16, a16, 0)

            def pe(i, _):
                k0g = q * KSUB + i * 16
                k0 = i * 16
                rows = iot + k0
                f0v = fb[0, pl.ds(k0g, 16)]
                f1v = fb[1, pl.ds(k0g, 16)]
                exv = exb[pl.ds(k0, 16)]
                for t in range(c):
                    gcol = plsc.load_gather(grows, [rows, colv[t]])
                    m = jnp.maximum(gcol + f0v * b0s[t] + f1v * b1s[t] + cvs[t],
                                    0.0)
                    plsc.store_scatter(msgs, [rows, colv[t]], m * exv)
                return 0
            lax.fori_loop(0, KSUB // 16, pe, 0)

        def outer(j, _):
            brow = wid * nout + j
            pltpu.sync_copy(ints_h.at[brow], ib)
            pltpu.sync_copy(flts_h.at[brow], fb)

            gd = {}
            sd = {}
            fill_didx(0, didx_l[0])
            gd[0] = start_gather(0, 0)
            for q in range(NSUB):
                cur = q % 3
                if q >= 2:
                    sd[q - 2].wait()
                if q + 1 < NSUB:
                    nxt = (q + 1) % 3
                    fill_didx(q + 1, didx_l[nxt])
                    gd[q + 1] = start_gather(q + 1, nxt)
                gd[q].wait()
                compute(q, cur)
                sd[q] = start_scatter(cur)
            sd[NSUB - 2].wait()
            sd[NSUB - 1].wait()
            return 0
        lax.fori_loop(0, nout, outer, 0)

        plsc.subcore_barrier()
        pltpu.sync_copy(acc_sp.at[pl.ds(sid * npt, npt)],
                        out_h.at[cid, pl.ds(sid * npt, npt)])

    return pl.kernel(
        body,
        out_type=jax.ShapeDtypeStruct((NC, NPAD, cw), jnp.float32),
        mesh=mesh,
        compiler_params=pltpu.CompilerParams(needs_layout_passes=False,
                                             use_tc_tiling_on_sc=False),
        scratch_types=[
            pltpu.VMEM_SHARED((NPAD, cw), jnp.float32),
            pltpu.VMEM_SHARED((NPAD, gp), jnp.float32),
            pltpu.VMEM((n,), jnp.float32),
            pltpu.VMEM((n,), jnp.float32),
            pltpu.VMEM((2, KB), jnp.int32),
            pltpu.VMEM((4, KB), jnp.float32),
            pltpu.VMEM((KSUB, gp), jnp.float32),
            pltpu.VMEM((KSUB, gp), jnp.float32),
            pltpu.VMEM((KSUB, gp), jnp.float32),
            pltpu.VMEM((KSUB, cw), jnp.float32),
            pltpu.VMEM((KSUB, cw), jnp.float32),
            pltpu.VMEM((KSUB, cw), jnp.float32),
            pltpu.VMEM((KSUB,), jnp.int32),
            pltpu.VMEM((KSUB,), jnp.int32),
            pltpu.VMEM((KSUB,), jnp.int32),
            pltpu.VMEM((KSUB,), jnp.float32),
            pltpu.VMEM((8, 48), jnp.float32),
            pltpu.SemaphoreType.DMA,
            pltpu.SemaphoreType.DMA,
            pltpu.SemaphoreType.DMA,
            pltpu.SemaphoreType.DMA,
            pltpu.SemaphoreType.DMA,
            pltpu.SemaphoreType.DMA,
        ],
    )


def kernel(x, edge_index, edge_attr, edge_feature, params):
    n = x.shape[0]
    e = edge_attr.shape[0]
    nblk = e // KB
    ints = jnp.stack([edge_index[0].reshape(nblk, KB),
                      edge_index[1].reshape(nblk, KB)], axis=1)
    flts = jnp.stack([edge_feature[:, 0].reshape(nblk, KB),
                      edge_feature[:, 1].reshape(nblk, KB),
                      edge_attr[:, 0].reshape(nblk, KB),
                      edge_attr[:, 1].reshape(nblk, KB)], axis=1)
    mea = _edge_absmax(edge_attr.reshape(-1, 512))[0, 0]

    h = x
    la = jnp.zeros((n, 2), jnp.float32)
    for li, name in enumerate(('mp1', 'mp2', 'mp3')):
        p = params[name]
        first = li == 0
        relu = li < 2
        c = p['att_src'].shape[-1]
        cw = c + 17
        a_mat = p['W_en'][:c, :]
        b8 = p['W_en'][c:, :]
        bf = p['W_ef'] @ b8
        cvec = p['b_ef'] @ b8 + p['b_en']
        v = p['W_edge'] @ p['att_edge'][0, 0, :]
        att2 = jnp.stack([p['att_src'][0, 0, :], p['att_dst'][0, 0, :]])

        pmat, gmat, ab, absum, mab = _node_pre(h, p['W_src'], a_mat, att2)
        m = jnp.maximum(mab[0, 0] + mea * (jnp.abs(v[0]) + jnp.abs(v[1])), 0.0)
        wv = (jnp.zeros((8, 48), jnp.float32)
              .at[0, :c].set(bf[0])
              .at[1, :c].set(bf[1])
              .at[2, :c].set(cvec)
              .at[3, 0].set(m)
              .at[3, 1].set(v[0])
              .at[3, 2].set(v[1]))
        zblk = jnp.zeros((NPAD // NS, cw), jnp.float32)

        acc2 = _make_sc(n, e, c, first)(
            ints, flts, gmat, ab[0], ab[1], wv, zblk)

        wrow = jnp.zeros((1, 8), jnp.float32).at[0, 0].set(v[0]).at[0, 1].set(v[1]).at[0, 2].set(m)
        bias = p['bias'].reshape(1, c)
        res = _node_post(acc2, pmat, absum, la, wrow, bias, c, cw, first, relu)
        if first:
            h, la = res
        else:
            h = res[0]
    return h


# speedup vs baseline: 1.5034x; 1.1861x over previous
import functools

import jax
import jax.numpy as jnp
from jax import lax
from jax.experimental import pallas as pl
from jax.experimental.pallas import tpu as pltpu
from jax.experimental.pallas import tpu_sc as plsc

NC = 2
NS = 16
NW = NC * NS
KSUB = 80
KB = 800
NSUB = KB // KSUB


def _node_pre(h, w_src, a_mat, att2):
    n = h.shape[0]
    c = w_src.shape[1]

    def body(h_ref, w_ref, a_ref, att_ref, p_ref, g_ref, ab_ref, abs_ref, mab_ref):
        hh = h_ref[...]
        p = jnp.dot(hh, w_ref[...], preferred_element_type=jnp.float32,
                    precision=lax.Precision.HIGHEST)
        p_ref[...] = p
        g_ref[...] = jnp.dot(p, a_ref[...], preferred_element_type=jnp.float32,
                             precision=lax.Precision.HIGHEST)
        att = att_ref[...]
        asrc = jnp.sum(p * att[0:1, :], axis=1)
        adst = jnp.sum(p * att[1:2, :], axis=1)
        ab_ref[...] = jnp.stack([asrc, adst])
        abs_ref[...] = (asrc + adst)[:, None]
        mab_ref[...] = (jnp.max(asrc) + jnp.max(adst)).reshape(1, 1)

    return pl.pallas_call(
        body,
        out_shape=[
            jax.ShapeDtypeStruct((n, c), jnp.float32),
            jax.ShapeDtypeStruct((n, c), jnp.float32),
            jax.ShapeDtypeStruct((2, n), jnp.float32),
            jax.ShapeDtypeStruct((n, 1), jnp.float32),
            jax.ShapeDtypeStruct((1, 1), jnp.float32),
        ],
    )(h, w_src, a_mat, att2)


def _edge_absmax(ea_rows):
    def body(ea_ref, out_ref):
        out_ref[...] = jnp.max(jnp.abs(ea_ref[...])).reshape(1, 1)

    return pl.pallas_call(
        body, out_shape=jax.ShapeDtypeStruct((1, 1), jnp.float32),
    )(ea_rows)


def _node_post(acc2, p, absum, la, wrow, bias, c, cw, first, relu):
    n = p.shape[0]

    def body(acc_ref, p_ref, abs_ref, la_ref, w_ref, b_ref, out_ref, la_out_ref=None):
        a = acc_ref[...]
        a = a[0, :n] + a[1, :n]
        msg = a[:, 0:c]
        s = a[:, c:c + 1]
        if first:
            cnt = a[:, c + 3:c + 4]
            ls = a[:, c + 1:c + 3]
            lav = jnp.where(cnt > 0, ls / jnp.maximum(cnt, 1.0), 0.0)
            la_out_ref[...] = lav
        else:
            lav = la_ref[...]
        w = w_ref[...]
        av = lav[:, 0:1] * w[0, 0] + lav[:, 1:2] * w[0, 1]
        al = abs_ref[...] + av
        al = jnp.maximum(al, 0.2 * al)
        exl = jnp.exp(al - w[0, 2])
        o = (msg + exl * p_ref[...]) / (s + exl + 1e-16) + b_ref[...]
        out_ref[...] = jnp.maximum(o, 0.0) if relu else o

    outs = [jax.ShapeDtypeStruct((n, c), jnp.float32)]
    if first:
        outs.append(jax.ShapeDtypeStruct((n, 2), jnp.float32))
    return pl.pallas_call(body, out_shape=outs)(acc2, p, absum, la, wrow, bias)


NPAD = 10240


@functools.lru_cache(maxsize=None)
def _make_sc(n, e, c, first):
    cw = c + 16
    epw = e // NW
    nout = epw // KB
    npt = NPAD // NS
    nch = c // 16
    mesh = plsc.VectorSubcoreMesh(core_axis_name="c", subcore_axis_name="s")

    nst = n // NS

    def body(ints_h, flts_h, g_h, asrc_h, adst_h, wv_h, z_h, out_h,
             acc_sp, g_sp, asrc_t, adst_t, ib, fb,
             grows0, grows1, grows2, msgs0, msgs1, msgs2,
             didx0, didx1, didx2, exb, wvb,
             gsem0, gsem1, gsem2, ssem0, ssem1, ssem2):
        grows_l = (grows0, grows1, grows2)
        msgs_l = (msgs0, msgs1, msgs2)
        didx_l = (didx0, didx1, didx2)
        gsem_l = (gsem0, gsem1, gsem2)
        ssem_l = (ssem0, ssem1, ssem2)
        cid = lax.axis_index("c")
        sid = lax.axis_index("s")
        wid = sid * NC + cid
        pltpu.sync_copy(asrc_h, asrc_t)
        pltpu.sync_copy(adst_h, adst_t)
        pltpu.sync_copy(wv_h, wvb)
        pltpu.sync_copy(z_h, acc_sp.at[pl.ds(sid * npt, npt)])
        pltpu.sync_copy(g_h.at[pl.ds(sid * nst, nst)],
                        g_sp.at[pl.ds(sid * nst, nst)])

        w3 = wvb[3, pl.ds(0, 16)]
        mv = w3[0]
        v0 = w3[1]
        v1 = w3[2]
        b0v = [wvb[0, pl.ds(16 * t, 16)] for t in range(nch)]
        b1v = [wvb[1, pl.ds(16 * t, 16)] for t in range(nch)]
        cvv = [wvb[2, pl.ds(16 * t, 16)] for t in range(nch)]
        z16 = jnp.zeros((16,), jnp.float32)

        def zrow(r, _):
            for mref in msgs_l:
                for t in range(cw // 16):
                    mref[r, pl.ds(16 * t, 16)] = z16
            return 0
        lax.fori_loop(0, KSUB, zrow, 0)

        plsc.subcore_barrier()

        iot = lax.iota(jnp.int32, 16)
        col_c = jnp.full((16,), c, jnp.int32)
        one16 = jnp.full((16,), 1.0, jnp.float32)

        def fill_didx(q, dref):
            for r in range(KSUB // 16):
                dref[pl.ds(r * 16, 16)] = ib[1, pl.ds(q * KSUB + r * 16, 16)]

        def start_gather(q, bi):
            return pltpu.async_copy(g_sp.at[ib.at[0, pl.ds(q * KSUB, KSUB)]],
                                    grows_l[bi], gsem_l[bi])

        def start_scatter(bi):
            return pltpu.async_copy(msgs_l[bi], acc_sp.at[didx_l[bi]],
                                    ssem_l[bi], add=True)

        def compute(q, bi):
            grows = grows_l[bi]
            msgs = msgs_l[bi]
            didx = didx_l[bi]

            def a16(i, _):
                k0g = q * KSUB + i * 16
                k0 = i * 16
                s16 = ib[0, pl.ds(k0g, 16)]
                d16 = didx[pl.ds(k0, 16)]
                a1 = plsc.load_gather(asrc_t, [s16])
                a2 = plsc.load_gather(adst_t, [d16])
                e0 = fb[2, pl.ds(k0g, 16)]
                e1 = fb[3, pl.ds(k0g, 16)]
                al = a1 + a2 + e0 * v0 + e1 * v1
                al = jnp.maximum(al, 0.2 * al)
                ex = jnp.exp(al - mv)
                exb[pl.ds(k0, 16)] = ex
                rows = iot + k0
                plsc.store_scatter(msgs, [rows, col_c], ex)
                if first:
                    plsc.store_scatter(msgs, [rows, col_c + 1], e0)
                    plsc.store_scatter(msgs, [rows, col_c + 2], e1)
                    plsc.store_scatter(msgs, [rows, col_c + 3], one16)
                return 0
            lax.fori_loop(0, KSUB // 16, a16, 0)

            def pe(i, _):
                k0g = q * KSUB + i * 16
                k0 = i * 16
                f0v = fb[0, pl.ds(k0g, 16)]
                f1v = fb[1, pl.ds(k0g, 16)]
                exv = exb[pl.ds(k0, 16)]
                for kk in range(16):
                    k = k0 + kk
                    f0 = f0v[kk]
                    f1 = f1v[kk]
                    ex = exv[kk]
                    for t in range(nch):
                        g = grows[k, pl.ds(16 * t, 16)]
                        m = jnp.maximum(g + f0 * b0v[t] + f1 * b1v[t] + cvv[t],
                                        0.0)
                        msgs[k, pl.ds(16 * t, 16)] = m * ex
                return 0
            lax.fori_loop(0, KSUB // 16, pe, 0)

        def outer(j, _):
            brow = wid * nout + j
            pltpu.sync_copy(ints_h.at[brow], ib)
            pltpu.sync_copy(flts_h.at[brow], fb)

            gd = {}
            sd = {}
            fill_didx(0, didx_l[0])
            gd[0] = start_gather(0, 0)
            for q in range(NSUB):
                cur = q % 3
                if q >= 2:
                    sd[q - 2].wait()
                if q + 1 < NSUB:
                    nxt = (q + 1) % 3
                    fill_didx(q + 1, didx_l[nxt])
                    gd[q + 1] = start_gather(q + 1, nxt)
                gd[q].wait()
                compute(q, cur)
                sd[q] = start_scatter(cur)
            sd[NSUB - 2].wait()
            sd[NSUB - 1].wait()
            return 0
        lax.fori_loop(0, nout, outer, 0)

        plsc.subcore_barrier()
        pltpu.sync_copy(acc_sp.at[pl.ds(sid * npt, npt)],
                        out_h.at[cid, pl.ds(sid * npt, npt)])

    return pl.kernel(
        body,
        out_type=jax.ShapeDtypeStruct((NC, NPAD, cw), jnp.float32),
        mesh=mesh,
        compiler_params=pltpu.CompilerParams(needs_layout_passes=False,
                                             use_tc_tiling_on_sc=False),
        scratch_types=[
            pltpu.VMEM_SHARED((NPAD, cw), jnp.float32),
            pltpu.VMEM_SHARED((n, c), jnp.float32),
            pltpu.VMEM((n,), jnp.float32),
            pltpu.VMEM((n,), jnp.float32),
            pltpu.VMEM((2, KB), jnp.int32),
            pltpu.VMEM((4, KB), jnp.float32),
            pltpu.VMEM((KSUB, c), jnp.float32),
            pltpu.VMEM((KSUB, c), jnp.float32),
            pltpu.VMEM((KSUB, c), jnp.float32),
            pltpu.VMEM((KSUB, cw), jnp.float32),
            pltpu.VMEM((KSUB, cw), jnp.float32),
            pltpu.VMEM((KSUB, cw), jnp.float32),
            pltpu.VMEM((KSUB,), jnp.int32),
            pltpu.VMEM((KSUB,), jnp.int32),
            pltpu.VMEM((KSUB,), jnp.int32),
            pltpu.VMEM((KSUB,), jnp.float32),
            pltpu.VMEM((8, 48), jnp.float32),
            pltpu.SemaphoreType.DMA,
            pltpu.SemaphoreType.DMA,
            pltpu.SemaphoreType.DMA,
            pltpu.SemaphoreType.DMA,
            pltpu.SemaphoreType.DMA,
            pltpu.SemaphoreType.DMA,
        ],
    )


def kernel(x, edge_index, edge_attr, edge_feature, params):
    n = x.shape[0]
    e = edge_attr.shape[0]
    nblk = e // KB
    ints = jnp.stack([edge_index[0].reshape(nblk, KB),
                      edge_index[1].reshape(nblk, KB)], axis=1)
    flts = jnp.stack([edge_feature[:, 0].reshape(nblk, KB),
                      edge_feature[:, 1].reshape(nblk, KB),
                      edge_attr[:, 0].reshape(nblk, KB),
                      edge_attr[:, 1].reshape(nblk, KB)], axis=1)
    mea = _edge_absmax(edge_attr.reshape(-1, 512))[0, 0]

    h = x
    la = jnp.zeros((n, 2), jnp.float32)
    for li, name in enumerate(('mp1', 'mp2', 'mp3')):
        p = params[name]
        first = li == 0
        relu = li < 2
        c = p['att_src'].shape[-1]
        cw = c + 16
        a_mat = p['W_en'][:c, :]
        b8 = p['W_en'][c:, :]
        bf = p['W_ef'] @ b8
        cvec = p['b_ef'] @ b8 + p['b_en']
        v = p['W_edge'] @ p['att_edge'][0, 0, :]
        att2 = jnp.stack([p['att_src'][0, 0, :], p['att_dst'][0, 0, :]])

        pmat, gmat, ab, absum, mab = _node_pre(h, p['W_src'], a_mat, att2)
        m = jnp.maximum(mab[0, 0] + mea * (jnp.abs(v[0]) + jnp.abs(v[1])), 0.0)
        wv = (jnp.zeros((8, 48), jnp.float32)
              .at[0, :c].set(bf[0])
              .at[1, :c].set(bf[1])
              .at[2, :c].set(cvec)
              .at[3, 0].set(m)
              .at[3, 1].set(v[0])
              .at[3, 2].set(v[1]))
        zblk = jnp.zeros((NPAD // NS, cw), jnp.float32)

        acc2 = _make_sc(n, e, c, first)(
            ints, flts, gmat, ab[0], ab[1], wv, zblk)

        wrow = jnp.zeros((1, 8), jnp.float32).at[0, 0].set(v[0]).at[0, 1].set(v[1]).at[0, 2].set(m)
        bias = p['bias'].reshape(1, c)
        res = _node_post(acc2, pmat, absum, la, wrow, bias, c, cw, first, relu)
        if first:
            h, la = res
        else:
            h = res[0]
    return h
